# baseline (speedup 1.0000x reference)
import functools

import jax
import jax.numpy as jnp
from jax import lax
from jax.experimental import pallas as pl
from jax.experimental.pallas import tpu as pltpu
from jax.experimental.pallas import tpu_sc as plsc

D_MODEL = 64
NUM_CORES = 2
NUM_SUBCORES = 16
NUM_WORKERS = NUM_CORES * NUM_SUBCORES
CHUNK = 128
GROUP = 5

VOCAB_N = 1000000
NBLK = VOCAB_N // 128 + 1
BLK_PER_W = (NBLK - 1) // NUM_WORKERS
NEXTRA = NBLK - BLK_PER_W * NUM_WORKERS


def _mesh():
    return plsc.VectorSubcoreMesh(core_axis_name="c", subcore_axis_name="s")


@jax.jit
def _format_table(table_t, tail_pairs):
    @functools.partial(
        pl.kernel,
        out_type=jax.ShapeDtypeStruct((VOCAB_N // 2, 128), jnp.float32),
        mesh=_mesh(),
        scratch_types=[
            pltpu.VMEM((D_MODEL, 128), jnp.float32),
            pltpu.VMEM((D_MODEL, 128), jnp.float32),
            pltpu.VMEM((D_MODEL, 128), jnp.float32),
            pltpu.VMEM((D_MODEL, 128), jnp.float32),
            pltpu.SemaphoreType.DMA,
            pltpu.SemaphoreType.DMA,
            pltpu.SemaphoreType.DMA,
            pltpu.SemaphoreType.DMA,
        ],
        compiler_params=pltpu.CompilerParams(
            use_tc_tiling_on_sc=True, needs_layout_passes=False),
    )
    def kern(tt_hbm, tail_hbm, scr_hbm, in_a, in_b, out_a, out_b,
             isem_a, isem_b, osem_a, osem_b):
        wid = lax.axis_index("s") * NUM_CORES + lax.axis_index("c")

        def in_start(blk, buf, sem):
            off = pl.multiple_of(blk * 128, 128)
            pltpu.async_copy(tt_hbm.at[:, pl.ds(off, 128)], buf, sem)

        def in_wait(buf, sem):
            pltpu.make_async_copy(tt_hbm.at[:, pl.ds(0, 128)], buf, sem).wait()

        def out_start(blk, buf, sem):
            off = pl.multiple_of(blk * 64, 64)
            pltpu.async_copy(buf, scr_hbm.at[pl.ds(off, 64), :], sem)

        def out_wait(buf, sem):
            pltpu.make_async_copy(buf, scr_hbm.at[pl.ds(0, 64), :],
                                  sem).wait()

        d_base = [
            lax.iota(jnp.int32, 16) + jnp.int32(16 * gg) for gg in range(4)
        ]

        def transpose_block(src, dst, nrows, t_off):
            def urow(u, carry):
                for g in range(8):
                    t = t_off + 2 * u + (g // 4)
                    vec = plsc.load_gather(
                        src, [d_base[g % 4], jnp.full((16,), t, jnp.int32)])
                    dst[u, pl.ds(16 * (g % 4) + 64 * (g // 4), 16)] = vec
                return carry

            lax.fori_loop(0, nrows, urow, 0)

        in_start(wid, in_a, isem_a)
        in_start(wid + NUM_WORKERS, in_b, isem_b)

        def body(p, carry):
            i_a, i_b = 2 * p, 2 * p + 1
            blk_a = wid + i_a * NUM_WORKERS
            blk_b = wid + i_b * NUM_WORKERS

            @pl.when(p > 0)
            def _():
                out_wait(out_a, osem_a)

            in_wait(in_a, isem_a)
            transpose_block(in_a, out_a, D_MODEL, 0)
            out_start(blk_a, out_a, osem_a)

            @pl.when(p < BLK_PER_W // 2 - 1)
            def _():
                in_start(blk_a + 2 * NUM_WORKERS, in_a, isem_a)

            @pl.when(p > 0)
            def _():
                out_wait(out_b, osem_b)

            in_wait(in_b, isem_b)
            transpose_block(in_b, out_b, D_MODEL, 0)
            out_start(blk_b, out_b, osem_b)

            @pl.when(p < BLK_PER_W // 2 - 1)
            def _():
                in_start(blk_b + 2 * NUM_WORKERS, in_b, isem_b)

            return carry

        lax.fori_loop(0, BLK_PER_W // 2, body, 0)
        out_wait(out_a, osem_a)
        out_wait(out_b, osem_b)

        @pl.when(wid < NEXTRA - 1)
        def _():
            blk = BLK_PER_W * NUM_WORKERS + wid
            in_start(blk, in_a, isem_a)
            in_wait(in_a, isem_a)
            transpose_block(in_a, out_a, D_MODEL, 0)
            out_start(blk, out_a, osem_a)
            out_wait(out_a, osem_a)

        @pl.when(wid == NEXTRA - 1)
        def _():
            pltpu.async_copy(tail_hbm, out_a.at[pl.ds(0, 32), :], isem_a)
            pltpu.make_async_copy(tail_hbm, out_a.at[pl.ds(0, 32), :],
                                  isem_a).wait()
            pltpu.async_copy(out_a.at[pl.ds(0, 32), :],
                             scr_hbm.at[pl.ds(VOCAB_N // 2 - 32, 32), :],
                             osem_a)
            pltpu.make_async_copy(out_a.at[pl.ds(0, 32), :],
                                  scr_hbm.at[pl.ds(0, 32), :], osem_a).wait()

    return kern(table_t, tail_pairs)


@jax.jit
def _embed(table, xt):
    seq, batch = xt.shape
    ngroups = seq // GROUP
    assert ngroups % 2 == 0 and ngroups >= 4
    assert batch == CHUNK * NUM_WORKERS

    @functools.partial(
        pl.kernel,
        out_type=jax.ShapeDtypeStruct((batch, seq, D_MODEL), jnp.float32),
        mesh=_mesh(),
        scratch_types=[
            pltpu.VMEM((seq, CHUNK), jnp.int32),
            pltpu.VMEM((GROUP * CHUNK, D_MODEL), jnp.float32),
            pltpu.VMEM((GROUP * CHUNK, D_MODEL), jnp.float32),
            pltpu.SemaphoreType.DMA,
            pltpu.SemaphoreType.DMA,
            pltpu.SemaphoreType.DMA,
            pltpu.SemaphoreType.DMA,
        ],
        compiler_params=pltpu.CompilerParams(use_tc_tiling_on_sc=False),
    )
    def kern(table_hbm, xt_hbm, out_hbm, idx_v, rows_a, rows_b,
             gsem_a, gsem_b, wsem_a, wsem_b):
        wid = lax.axis_index("s") * NUM_CORES + lax.axis_index("c")
        base_b = wid * CHUNK
        pltpu.sync_copy(xt_hbm.at[:, pl.ds(base_b, CHUNK)], idx_v)

        def gathers_start(g, buf, sem):
            for i in range(GROUP):
                pltpu.async_copy(
                    table_hbm.at[idx_v.at[g * GROUP + i]],
                    buf.at[pl.ds(i * CHUNK, CHUNK)], sem)

        def gathers_wait(buf, sem):
            for i in range(GROUP):
                pltpu.make_async_copy(
                    table_hbm.at[idx_v.at[0]],
                    buf.at[pl.ds(i * CHUNK, CHUNK)], sem).wait()

        def writes_start(g, buf, sem):
            for i in range(GROUP):
                pltpu.async_copy(
                    buf.at[pl.ds(i * CHUNK, CHUNK)],
                    out_hbm.at[pl.ds(base_b, CHUNK), g * GROUP + i], sem)

        def writes_wait(buf, sem):
            for i in range(GROUP):
                pltpu.make_async_copy(
                    buf.at[pl.ds(i * CHUNK, CHUNK)],
                    out_hbm.at[pl.ds(base_b, CHUNK), 0], sem).wait()

        gathers_start(0, rows_a, gsem_a)
        gathers_start(1, rows_b, gsem_b)
        gathers_wait(rows_a, gsem_a)
        writes_start(0, rows_a, wsem_a)

        def body(p, carry):
            ga = 2 * p + 2
            writes_wait(rows_a, wsem_a)
            gathers_start(ga, rows_a, gsem_a)
            gathers_wait(rows_b, gsem_b)
            writes_start(ga - 1, rows_b, wsem_b)
            writes_wait(rows_b, wsem_b)
            gathers_start(ga + 1, rows_b, gsem_b)
            gathers_wait(rows_a, gsem_a)
            writes_start(ga, rows_a, wsem_a)
            return carry

        lax.fori_loop(0, (ngroups - 2) // 2, body, 0)
        gathers_wait(rows_b, gsem_b)
        writes_start(ngroups - 1, rows_b, wsem_b)
        writes_wait(rows_a, wsem_a)
        writes_wait(rows_b, wsem_b)

    return kern(table, xt)


COLS_PER_BLK = 4096


@jax.jit
def _format_table_tc(table_t):
    nblk = (VOCAB_N + COLS_PER_BLK - 1) // COLS_PER_BLK

    def body(in_ref, out_ref):
        blk = in_ref[...]
        t = jnp.transpose(blk, (1, 0))
        t3 = t.reshape(COLS_PER_BLK // 2, 2, D_MODEL)
        out_ref[...] = jnp.concatenate([t3[:, 0, :], t3[:, 1, :]], axis=1)

    return pl.pallas_call(
        body,
        grid=(nblk,),
        in_specs=[pl.BlockSpec((D_MODEL, COLS_PER_BLK), lambda i: (0, i))],
        out_specs=pl.BlockSpec((COLS_PER_BLK // 2, 128), lambda i: (i, 0)),
        out_shape=jax.ShapeDtypeStruct((VOCAB_N // 2, 128), jnp.float32),
        compiler_params=pltpu.CompilerParams(
            dimension_semantics=("parallel",)),
    )(table_t)


def kernel(x, table):
    table_t = jnp.swapaxes(table, 0, 1)
    scratch = _format_table_tc(table_t)
    rowmajor = jnp.reshape(scratch, (VOCAB_N, D_MODEL))
    xt = jnp.swapaxes(x, 0, 1).astype(jnp.int32)
    return _embed(rowmajor, xt)


# speedup vs baseline: 1.0417x; 1.0417x over previous
import functools

import jax
import jax.numpy as jnp
from jax import lax
from jax.experimental import pallas as pl
from jax.experimental.pallas import tpu as pltpu
from jax.experimental.pallas import tpu_sc as plsc

D_MODEL = 64
NUM_CORES = 2
NUM_SUBCORES = 16
NUM_WORKERS = NUM_CORES * NUM_SUBCORES
CHUNK = 128
GROUP = 5

VOCAB_N = 1000000
NBLK = VOCAB_N // 128 + 1
BLK_PER_W = (NBLK - 1) // NUM_WORKERS
NEXTRA = NBLK - BLK_PER_W * NUM_WORKERS


def _mesh():
    return plsc.VectorSubcoreMesh(core_axis_name="c", subcore_axis_name="s")


@jax.jit
def _format_table(table_t, tail_pairs):
    @functools.partial(
        pl.kernel,
        out_type=jax.ShapeDtypeStruct((VOCAB_N // 2, 128), jnp.float32),
        mesh=_mesh(),
        scratch_types=[
            pltpu.VMEM((D_MODEL, 128), jnp.float32),
            pltpu.VMEM((D_MODEL, 128), jnp.float32),
            pltpu.VMEM((D_MODEL, 128), jnp.float32),
            pltpu.VMEM((D_MODEL, 128), jnp.float32),
            pltpu.SemaphoreType.DMA,
            pltpu.SemaphoreType.DMA,
            pltpu.SemaphoreType.DMA,
            pltpu.SemaphoreType.DMA,
        ],
        compiler_params=pltpu.CompilerParams(
            use_tc_tiling_on_sc=True, needs_layout_passes=False),
    )
    def kern(tt_hbm, tail_hbm, scr_hbm, in_a, in_b, out_a, out_b,
             isem_a, isem_b, osem_a, osem_b):
        wid = lax.axis_index("s") * NUM_CORES + lax.axis_index("c")

        def in_start(blk, buf, sem):
            off = pl.multiple_of(blk * 128, 128)
            pltpu.async_copy(tt_hbm.at[:, pl.ds(off, 128)], buf, sem)

        def in_wait(buf, sem):
            pltpu.make_async_copy(tt_hbm.at[:, pl.ds(0, 128)], buf, sem).wait()

        def out_start(blk, buf, sem):
            off = pl.multiple_of(blk * 64, 64)
            pltpu.async_copy(buf, scr_hbm.at[pl.ds(off, 64), :], sem)

        def out_wait(buf, sem):
            pltpu.make_async_copy(buf, scr_hbm.at[pl.ds(0, 64), :],
                                  sem).wait()

        d_base = [
            lax.iota(jnp.int32, 16) + jnp.int32(16 * gg) for gg in range(4)
        ]

        def transpose_block(src, dst, nrows, t_off):
            def urow(u, carry):
                for g in range(8):
                    t = t_off + 2 * u + (g // 4)
                    vec = plsc.load_gather(
                        src, [d_base[g % 4], jnp.full((16,), t, jnp.int32)])
                    dst[u, pl.ds(16 * (g % 4) + 64 * (g // 4), 16)] = vec
                return carry

            lax.fori_loop(0, nrows, urow, 0)

        in_start(wid, in_a, isem_a)
        in_start(wid + NUM_WORKERS, in_b, isem_b)

        def body(p, carry):
            i_a, i_b = 2 * p, 2 * p + 1
            blk_a = wid + i_a * NUM_WORKERS
            blk_b = wid + i_b * NUM_WORKERS

            @pl.when(p > 0)
            def _():
                out_wait(out_a, osem_a)

            in_wait(in_a, isem_a)
            transpose_block(in_a, out_a, D_MODEL, 0)
            out_start(blk_a, out_a, osem_a)

            @pl.when(p < BLK_PER_W // 2 - 1)
            def _():
                in_start(blk_a + 2 * NUM_WORKERS, in_a, isem_a)

            @pl.when(p > 0)
            def _():
                out_wait(out_b, osem_b)

            in_wait(in_b, isem_b)
            transpose_block(in_b, out_b, D_MODEL, 0)
            out_start(blk_b, out_b, osem_b)

            @pl.when(p < BLK_PER_W // 2 - 1)
            def _():
                in_start(blk_b + 2 * NUM_WORKERS, in_b, isem_b)

            return carry

        lax.fori_loop(0, BLK_PER_W // 2, body, 0)
        out_wait(out_a, osem_a)
        out_wait(out_b, osem_b)

        @pl.when(wid < NEXTRA - 1)
        def _():
            blk = BLK_PER_W * NUM_WORKERS + wid
            in_start(blk, in_a, isem_a)
            in_wait(in_a, isem_a)
            transpose_block(in_a, out_a, D_MODEL, 0)
            out_start(blk, out_a, osem_a)
            out_wait(out_a, osem_a)

        @pl.when(wid == NEXTRA - 1)
        def _():
            pltpu.async_copy(tail_hbm, out_a.at[pl.ds(0, 32), :], isem_a)
            pltpu.make_async_copy(tail_hbm, out_a.at[pl.ds(0, 32), :],
                                  isem_a).wait()
            pltpu.async_copy(out_a.at[pl.ds(0, 32), :],
                             scr_hbm.at[pl.ds(VOCAB_N // 2 - 32, 32), :],
                             osem_a)
            pltpu.make_async_copy(out_a.at[pl.ds(0, 32), :],
                                  scr_hbm.at[pl.ds(0, 32), :], osem_a).wait()

    return kern(table_t, tail_pairs)


@jax.jit
def _embed(table, xt):
    seq, batch = xt.shape
    ngroups = seq // GROUP
    assert ngroups % 2 == 0 and ngroups >= 4
    assert batch == CHUNK * NUM_WORKERS

    @functools.partial(
        pl.kernel,
        out_type=jax.ShapeDtypeStruct((batch, seq, D_MODEL), jnp.float32),
        mesh=_mesh(),
        scratch_types=[
            pltpu.VMEM((seq, CHUNK), jnp.int32),
            pltpu.VMEM((GROUP * CHUNK, D_MODEL), jnp.float32),
            pltpu.VMEM((GROUP * CHUNK, D_MODEL), jnp.float32),
            pltpu.SemaphoreType.DMA,
            pltpu.SemaphoreType.DMA,
            pltpu.SemaphoreType.DMA,
            pltpu.SemaphoreType.DMA,
        ],
        compiler_params=pltpu.CompilerParams(use_tc_tiling_on_sc=False),
    )
    def kern(table_hbm, xt_hbm, out_hbm, idx_v, rows_a, rows_b,
             gsem_a, gsem_b, wsem_a, wsem_b):
        wid = lax.axis_index("s") * NUM_CORES + lax.axis_index("c")
        base_b = wid * CHUNK
        pltpu.sync_copy(xt_hbm.at[:, pl.ds(base_b, CHUNK)], idx_v)

        def gathers_start(g, buf, sem):
            for i in range(GROUP):
                pltpu.async_copy(
                    table_hbm.at[idx_v.at[g * GROUP + i]],
                    buf.at[pl.ds(i * CHUNK, CHUNK)], sem)

        def gathers_wait(buf, sem):
            for i in range(GROUP):
                pltpu.make_async_copy(
                    table_hbm.at[idx_v.at[0]],
                    buf.at[pl.ds(i * CHUNK, CHUNK)], sem).wait()

        def writes_start(g, buf, sem):
            for i in range(GROUP):
                pltpu.async_copy(
                    buf.at[pl.ds(i * CHUNK, CHUNK)],
                    out_hbm.at[pl.ds(base_b, CHUNK), g * GROUP + i], sem)

        def writes_wait(buf, sem):
            for i in range(GROUP):
                pltpu.make_async_copy(
                    buf.at[pl.ds(i * CHUNK, CHUNK)],
                    out_hbm.at[pl.ds(base_b, CHUNK), 0], sem).wait()

        gathers_start(0, rows_a, gsem_a)
        gathers_start(1, rows_b, gsem_b)
        gathers_wait(rows_a, gsem_a)
        writes_start(0, rows_a, wsem_a)

        def body(p, carry):
            ga = 2 * p + 2
            writes_wait(rows_a, wsem_a)
            gathers_start(ga, rows_a, gsem_a)
            gathers_wait(rows_b, gsem_b)
            writes_start(ga - 1, rows_b, wsem_b)
            writes_wait(rows_b, wsem_b)
            gathers_start(ga + 1, rows_b, gsem_b)
            gathers_wait(rows_a, gsem_a)
            writes_start(ga, rows_a, wsem_a)
            return carry

        lax.fori_loop(0, (ngroups - 2) // 2, body, 0)
        gathers_wait(rows_b, gsem_b)
        writes_start(ngroups - 1, rows_b, wsem_b)
        writes_wait(rows_a, wsem_a)
        writes_wait(rows_b, wsem_b)

    return kern(table, xt)


COLS_PER_BLK = 16384


@jax.jit
def _format_table_tc(table_t):
    nblk = (VOCAB_N + COLS_PER_BLK - 1) // COLS_PER_BLK

    def body(in_ref, out_ref):
        blk = in_ref[...]
        t = jnp.transpose(blk, (1, 0))
        t3 = t.reshape(COLS_PER_BLK // 2, 2, D_MODEL)
        out_ref[...] = jnp.concatenate([t3[:, 0, :], t3[:, 1, :]], axis=1)

    return pl.pallas_call(
        body,
        grid=(nblk,),
        in_specs=[pl.BlockSpec((D_MODEL, COLS_PER_BLK), lambda i: (0, i))],
        out_specs=pl.BlockSpec((COLS_PER_BLK // 2, 128), lambda i: (i, 0)),
        out_shape=jax.ShapeDtypeStruct((VOCAB_N // 2, 128), jnp.float32),
        compiler_params=pltpu.CompilerParams(
            dimension_semantics=("parallel",)),
    )(table_t)


def kernel(x, table):
    table_t = jnp.swapaxes(table, 0, 1)
    scratch = _format_table_tc(table_t)
    rowmajor = jnp.reshape(scratch, (VOCAB_N, D_MODEL))
    xt = jnp.swapaxes(x, 0, 1).astype(jnp.int32)
    return _embed(rowmajor, xt)
